# Initial kernel scaffold; baseline (speedup 1.0000x reference)
import jax
import jax.numpy as jnp
from jax.experimental import pallas as pl


def kernel(pcd_prev, K_prev, e_p, e_x, e_o, params):
    raise NotImplementedError("write your pallas kernel here")


# speedup vs baseline: 5.1981x; 5.1981x over previous
import functools

import jax
import jax.numpy as jnp
from jax import lax
from jax.experimental import pallas as pl
from jax.experimental.pallas import tpu as pltpu
from jax.experimental.pallas import tpu_sc as plsc

_N = 10000
_SEG = 5000
_SEGP = 5120
_NS = 16
_ROWS = _N * _NS
_EPS = 1e-5
_KNN_R = 256
_PB = 400
_RB = _PB * _NS
_GB = _N // _PB
_INTERPRET = False


def _f32dot(a, b):
    return jnp.dot(a, b, preferred_element_type=jnp.float32)


def _front_body(kp, l1w, l1b, g1, b1, wq, bq, wk, bk, wv, bv,
                ko, xqo, xko, xvo):
    t = _f32dot(kp[...], l1w[...]) + l1b[...]
    m = jnp.mean(t, axis=0, keepdims=True)
    v = jnp.mean(jnp.square(t - m), axis=0, keepdims=True)
    k = jnp.maximum((t - m) / jnp.sqrt(v + _EPS) * g1[...] + b1[...], 0.0)
    ko[...] = k
    xqo[...] = _f32dot(k, wq[...]) + bq[...]
    xko[...] = _f32dot(k, wk[...]) + bk[...]
    xvo[...] = _f32dot(k, wv[...]) + bv[...]


def _front(kprev, l1w, l1b, g1, b1, wq, bq, wk, bk, wv, bv):
    o = jax.ShapeDtypeStruct((_N, 128), jnp.float32)
    return pl.pallas_call(
        _front_body,
        out_shape=[o, o, o, o],
        interpret=_INTERPRET,
    )(kprev, l1w, l1b, g1, b1, wq, bq, wk, bk, wv, bv)


def _knn_body(prow, pall, idxo):
    pr = prow[...]
    pa = pall[...]
    dots = lax.dot_general(pr, pa, (((1,), (1,)), ((), ())),
                           preferred_element_type=jnp.float32)
    sqa = jnp.sum(pa * pa, axis=1, keepdims=True)
    d = jnp.transpose(sqa) - 2.0 * dots
    col = lax.broadcasted_iota(jnp.int32, d.shape, 1)
    inf = jnp.float32(jnp.inf)
    d = jnp.where(col >= _SEG, inf, d)
    outs = []
    for _ in range(_NS):
        mval = jnp.min(d, axis=1, keepdims=True)
        cand = jnp.where(d <= mval, col, jnp.int32(1 << 30))
        sel = jnp.min(cand, axis=1, keepdims=True)
        outs.append(sel)
        d = jnp.where(col == sel, inf, d)
    idxo[...] = jnp.concatenate(outs, axis=1)


def _knn_segment(p_seg_padded):
    return pl.pallas_call(
        _knn_body,
        grid=(_SEGP // _KNN_R,),
        in_specs=[
            pl.BlockSpec((_KNN_R, 3), lambda i: (i, 0)),
            pl.BlockSpec((_SEGP, 3), lambda i: (0, 0)),
        ],
        out_specs=pl.BlockSpec((_KNN_R, _NS), lambda i: (i, 0)),
        out_shape=jax.ShapeDtypeStruct((_SEGP, _NS), jnp.int32),
        interpret=_INTERPRET,
    )(p_seg_padded, p_seg_padded)[:_SEG]


def _gather2(kv, pp, idx2):
    nchunks = idx2.shape[0]
    mesh = plsc.VectorSubcoreMesh(core_axis_name="c", subcore_axis_name="s")

    @functools.partial(
        pl.kernel,
        mesh=mesh,
        out_type=[
            jax.ShapeDtypeStruct((_ROWS, 256), jnp.float32),
            jax.ShapeDtypeStruct((_ROWS, 128), jnp.float32),
        ],
        scratch_types=[
            pltpu.VMEM((128,), jnp.int32),
            pltpu.VMEM((128, 256), jnp.float32),
            pltpu.VMEM((128, 128), jnp.float32),
            pltpu.SemaphoreType.DMA,
        ],
    )
    def gk(kv_hbm, pp_hbm, idx_hbm, okv_hbm, op_hbm,
           idx_v, kvb, pb, sem):
        wid = lax.axis_index("s") * 2 + lax.axis_index("c")
        extra = jnp.where(wid < nchunks % 32, 1, 0)
        nj = nchunks // 32 + extra

        def body(j, carry):
            c = wid + 32 * j
            pltpu.sync_copy(idx_hbm.at[c], idx_v)
            pltpu.async_copy(kv_hbm.at[idx_v], kvb, sem).wait()
            pltpu.sync_copy(kvb, okv_hbm.at[pl.ds(c * 128, 128)])
            pltpu.async_copy(pp_hbm.at[idx_v], pb, sem).wait()
            pltpu.sync_copy(pb, op_hbm.at[pl.ds(c * 128, 128)])
            return carry

        lax.fori_loop(0, nj, body, 0)

    return gk(kv, pp, idx2)


def _prstats_body(pg, prep, w1, b1, so, s2o):
    gx = pg[...][:, :8] - prep[...]
    t = _f32dot(gx, w1[...]) + b1[...]

    @pl.when(pl.program_id(0) == 0)
    def _():
        so[...] = jnp.zeros_like(so)
        s2o[...] = jnp.zeros_like(s2o)

    so[...] += jnp.sum(t, axis=0, keepdims=True)
    s2o[...] += jnp.sum(t * t, axis=0, keepdims=True)


def _prstats(pg, prep, w1, b1):
    row = lambda i: (i, 0)
    fix = lambda i: (0, 0)
    o = jax.ShapeDtypeStruct((1, 8), jnp.float32)
    return pl.pallas_call(
        _prstats_body,
        grid=(_GB,),
        in_specs=[
            pl.BlockSpec((_RB, 128), row),
            pl.BlockSpec((_RB, 8), row),
            pl.BlockSpec((8, 8), fix),
            pl.BlockSpec((1, 8), fix),
        ],
        out_specs=[pl.BlockSpec((1, 8), fix), pl.BlockSpec((1, 8), fix)],
        out_shape=[o, o],
        interpret=_INTERPRET,
    )(pg, prep, w1, b1)


def _pr_block(pg, prep, w1, b1, prs, prs2, g, bb, w2, b2):
    gx = pg - prep
    t = _f32dot(gx, w1) + b1
    m = prs / jnp.float32(_ROWS)
    v = prs2 / jnp.float32(_ROWS) - m * m
    tn = jnp.maximum((t - m) / jnp.sqrt(v + _EPS) * g + bb, 0.0)
    return _f32dot(tn, w2) + b2


def _wpre_body(xkg, xqr, pg, prep, w1, b1, prs, prs2, g, bb, w2, b2,
               wpo, so, s2o):
    p_r = _pr_block(pg[...][:, :8], prep[...], w1[...], b1[...], prs[...],
                    prs2[...], g[...], bb[...], w2[...], b2[...])
    w = xkg[...] - xqr[...] + p_r
    wpo[...] = w

    @pl.when(pl.program_id(0) == 0)
    def _():
        so[...] = jnp.zeros_like(so)
        s2o[...] = jnp.zeros_like(s2o)

    so[...] += jnp.sum(w, axis=0, keepdims=True)
    s2o[...] += jnp.sum(w * w, axis=0, keepdims=True)


def _wpre(xkg, xqr, pg, prep, w1, b1, prs, prs2, g, bb, w2, b2):
    row = lambda i: (i, 0)
    fix = lambda i: (0, 0)
    return pl.pallas_call(
        _wpre_body,
        grid=(_GB,),
        in_specs=[
            pl.BlockSpec((_RB, 128), row),
            pl.BlockSpec((_RB, 128), row),
            pl.BlockSpec((_RB, 128), row),
            pl.BlockSpec((_RB, 8), row),
            pl.BlockSpec((8, 8), fix),
            pl.BlockSpec((1, 8), fix),
            pl.BlockSpec((1, 8), fix),
            pl.BlockSpec((1, 8), fix),
            pl.BlockSpec((1, 8), fix),
            pl.BlockSpec((1, 8), fix),
            pl.BlockSpec((8, 128), fix),
            pl.BlockSpec((1, 128), fix),
        ],
        out_specs=[
            pl.BlockSpec((_RB, 128), row),
            pl.BlockSpec((1, 128), fix),
            pl.BlockSpec((1, 128), fix),
        ],
        out_shape=[
            jax.ShapeDtypeStruct((_ROWS, 128), jnp.float32),
            jax.ShapeDtypeStruct((1, 128), jnp.float32),
            jax.ShapeDtypeStruct((1, 128), jnp.float32),
        ],
        interpret=_INTERPRET,
    )(xkg, xqr, pg, prep, w1, b1, prs, prs2, g, bb, w2, b2)


def _h1_body(wp, ws, ws2, g0, b0, w1, b1, h1o, so, s2o):
    m = ws[...] / jnp.float32(_ROWS)
    v = ws2[...] / jnp.float32(_ROWS) - m * m
    wn = jnp.maximum((wp[...] - m) / jnp.sqrt(v + _EPS) * g0[...] + b0[...], 0.0)
    h1 = _f32dot(wn, w1[...]) + b1[...]
    h1o[...] = h1

    @pl.when(pl.program_id(0) == 0)
    def _():
        so[...] = jnp.zeros_like(so)
        s2o[...] = jnp.zeros_like(s2o)

    so[...] += jnp.sum(h1, axis=0, keepdims=True)
    s2o[...] += jnp.sum(h1 * h1, axis=0, keepdims=True)


def _h1(wp, ws, ws2, g0, b0, w1, b1):
    row = lambda i: (i, 0)
    fix = lambda i: (0, 0)
    return pl.pallas_call(
        _h1_body,
        grid=(_GB,),
        in_specs=[
            pl.BlockSpec((_RB, 128), row),
            pl.BlockSpec((1, 128), fix),
            pl.BlockSpec((1, 128), fix),
            pl.BlockSpec((1, 128), fix),
            pl.BlockSpec((1, 128), fix),
            pl.BlockSpec((128, 16), fix),
            pl.BlockSpec((1, 16), fix),
        ],
        out_specs=[
            pl.BlockSpec((_RB, 16), row),
            pl.BlockSpec((1, 16), fix),
            pl.BlockSpec((1, 16), fix),
        ],
        out_shape=[
            jax.ShapeDtypeStruct((_ROWS, 16), jnp.float32),
            jax.ShapeDtypeStruct((1, 16), jnp.float32),
            jax.ShapeDtypeStruct((1, 16), jnp.float32),
        ],
        interpret=_INTERPRET,
    )(wp, ws, ws2, g0, b0, w1, b1)


def _agg_body(h1, hs, hs2, g1, bb1, w2, b2,
              gv, pg, prep, lpw1, lpb1, prs, prs2, lpg, lpbb, lpw2, lpb2,
              oo, so, s2o):
    m = hs[...] / jnp.float32(_ROWS)
    v = hs2[...] / jnp.float32(_ROWS) - m * m
    h = jnp.maximum((h1[...] - m) / jnp.sqrt(v + _EPS) * g1[...] + bb1[...], 0.0)
    w = _f32dot(h, w2[...]) + b2[...]
    w3 = w.reshape(_PB, _NS, _NS)
    mx = jnp.max(w3, axis=1, keepdims=True)
    e = jnp.exp(w3 - mx)
    sm = e / jnp.sum(e, axis=1, keepdims=True)
    p_r = _pr_block(pg[...][:, :8], prep[...], lpw1[...], lpb1[...], prs[...],
                    prs2[...], lpg[...], lpbb[...], lpw2[...], lpb2[...])
    val = (gv[...] + p_r).reshape(_PB, _NS, 128)
    wt = jnp.concatenate([sm] * 8, axis=2)
    ob = jnp.sum(val * wt, axis=1)
    oo[...] = ob

    @pl.when(pl.program_id(0) == 0)
    def _():
        so[...] = jnp.zeros_like(so)
        s2o[...] = jnp.zeros_like(s2o)

    so[...] += jnp.sum(ob, axis=0, keepdims=True)
    s2o[...] += jnp.sum(ob * ob, axis=0, keepdims=True)


def _agg(h1, hs, hs2, g1, bb1, w2, b2,
         gv, pg, prep, lpw1, lpb1, prs, prs2, lpg, lpbb, lpw2, lpb2):
    row = lambda i: (i, 0)
    fix = lambda i: (0, 0)
    return pl.pallas_call(
        _agg_body,
        grid=(_GB,),
        in_specs=[
            pl.BlockSpec((_RB, 16), row),
            pl.BlockSpec((1, 16), fix),
            pl.BlockSpec((1, 16), fix),
            pl.BlockSpec((1, 16), fix),
            pl.BlockSpec((1, 16), fix),
            pl.BlockSpec((16, 16), fix),
            pl.BlockSpec((1, 16), fix),
            pl.BlockSpec((_RB, 128), lambda i: (i, 1)),
            pl.BlockSpec((_RB, 128), row),
            pl.BlockSpec((_RB, 8), row),
            pl.BlockSpec((8, 8), fix),
            pl.BlockSpec((1, 8), fix),
            pl.BlockSpec((1, 8), fix),
            pl.BlockSpec((1, 8), fix),
            pl.BlockSpec((1, 8), fix),
            pl.BlockSpec((1, 8), fix),
            pl.BlockSpec((8, 128), fix),
            pl.BlockSpec((1, 128), fix),
        ],
        out_specs=[
            pl.BlockSpec((_PB, 128), row),
            pl.BlockSpec((1, 128), fix),
            pl.BlockSpec((1, 128), fix),
        ],
        out_shape=[
            jax.ShapeDtypeStruct((_N, 128), jnp.float32),
            jax.ShapeDtypeStruct((1, 128), jnp.float32),
            jax.ShapeDtypeStruct((1, 128), jnp.float32),
        ],
        interpret=_INTERPRET,
    )(h1, hs, hs2, g1, bb1, w2, b2,
      gv, pg, prep, lpw1, lpb1, prs, prs2, lpg, lpbb, lpw2, lpb2)


def _bn_rows(x, g, b):
    m = jnp.mean(x, axis=0, keepdims=True)
    v = jnp.mean(jnp.square(x - m), axis=0, keepdims=True)
    return (x - m) / jnp.sqrt(v + _EPS) * g + b


def _tail_body(st, kk, s2, s2q, g2, b2, psw1, psb1, psg, psbb, psw2, psb2,
               cw0, cw1, mdfw1, mdfb1, mdfg, mdfbb, mdfw2, mdfb2,
               scw, scb, scg, scbb, mdw1, mdb1, mdg, mdbb, mdw2, mdb2,
               pcd, pco, kco):
    m = s2[...] / jnp.float32(_N)
    v = s2q[...] / jnp.float32(_N) - m * m
    h = jnp.maximum((st[...] - m) / jnp.sqrt(v + _EPS) * g2[...] + b2[...], 0.0)
    h = h + kk[...]
    t1 = jnp.maximum(_bn_rows(_f32dot(h, psw1[...]) + psb1[...],
                              psg[...], psbb[...]), 0.0)
    fc = _f32dot(t1, psw2[...]) + psb2[...]
    a0 = _f32dot(fc, cw0[...])
    a1 = _f32dot(fc, cw1[...])
    feat = jnp.concatenate([a0, a1], axis=0)
    kc = jnp.maximum(_bn_rows(_f32dot(feat, mdfw1[...]) + mdfb1[...],
                              mdfg[...], mdfbb[...]), 0.0)
    kc = _f32dot(kc, mdfw2[...]) + mdfb2[...]
    sc_ = _bn_rows(_f32dot(feat, scw[...]) + scb[...], scg[...], scbb[...])
    k_curr = jnp.maximum(kc + sc_, 0.0)
    md = jnp.maximum(_bn_rows(_f32dot(k_curr, mdw1[...]) + mdb1[...],
                              mdg[...], mdbb[...]), 0.0)
    delta = jnp.tanh(jnp.tanh(_f32dot(md, mdw2[...]) + mdb2[...]))
    p = pcd[...]
    pco[...] = jnp.concatenate([p, p], axis=0) + delta
    kco[...] = k_curr


def _tail(st, kk, s2, s2q, g2, b2, psw1, psb1, psg, psbb, psw2, psb2,
          cw0, cw1, mdfw1, mdfb1, mdfg, mdfbb, mdfw2, mdfb2,
          scw, scb, scg, scbb, mdw1, mdb1, mdg, mdbb, mdw2, mdb2, pcd):
    return pl.pallas_call(
        _tail_body,
        out_shape=[
            jax.ShapeDtypeStruct((2 * _N, 3), jnp.float32),
            jax.ShapeDtypeStruct((2 * _N, 128), jnp.float32),
        ],
        interpret=_INTERPRET,
    )(st, kk, s2, s2q, g2, b2, psw1, psb1, psg, psbb, psw2, psb2,
      cw0, cw1, mdfw1, mdfb1, mdfg, mdfbb, mdfw2, mdfb2,
      scw, scb, scg, scbb, mdw1, mdb1, mdg, mdbb, mdw2, mdb2, pcd)


def _row(x):
    return jnp.asarray(x, jnp.float32).reshape(1, -1)


def _pad_lane(x, n):
    return jnp.pad(jnp.asarray(x, jnp.float32), ((0, 0), (0, n - x.shape[1])))


def kernel(pcd_prev, K_prev, e_p, e_x, e_o, params):
    P = params
    del e_p, e_x, e_o

    k, xq, xk, xv = _front(
        K_prev, P['l1_w'], _row(P['l1_b']), _row(P['bn1_g']), _row(P['bn1_b']),
        P['st_lq_w'], _row(P['st_lq_b']), P['st_lk_w'], _row(P['st_lk_b']),
        P['st_lv_w'], _row(P['st_lv_b']))

    p0 = jnp.pad(pcd_prev[:_SEG], ((0, _SEGP - _SEG), (0, 0)))
    p1 = jnp.pad(pcd_prev[_SEG:], ((0, _SEGP - _SEG), (0, 0)))
    idx0 = _knn_segment(p0)
    idx1 = _knn_segment(p1) + _SEG
    idx2 = jnp.concatenate([idx0, idx1], axis=0).reshape(_ROWS // 128, 128)

    kv = jnp.concatenate([xk, xv], axis=1)
    pp128 = jnp.pad(pcd_prev, ((0, 0), (0, 125)))
    kvg, pg = _gather2(kv, pp128, idx2)

    prep = jnp.repeat(pp128[:, :8], _NS, axis=0)
    xqr = jnp.repeat(xq, _NS, axis=0)

    lpw1 = jnp.zeros((8, 8), jnp.float32).at[:3, :3].set(P['st_lp_w1'])
    lpb1 = jnp.zeros((1, 8), jnp.float32).at[:, :3].set(P['st_lp_b1'])
    lpg = jnp.zeros((1, 8), jnp.float32).at[:, :3].set(P['st_lp_g'])
    lpbb = jnp.zeros((1, 8), jnp.float32).at[:, :3].set(P['st_lp_bb'])
    lpw2 = jnp.zeros((8, 128), jnp.float32).at[:3, :].set(P['st_lp_w2'])
    lpb2 = _row(P['st_lp_b2'])

    prs, prs2 = _prstats(pg, prep, lpw1, lpb1)

    wp, ws, ws2 = _wpre(kvg, xqr, pg, prep, lpw1, lpb1, prs, prs2,
                        lpg, lpbb, lpw2, lpb2)

    h1, hs, hs2 = _h1(wp, ws, ws2, _row(P['st_lw_g0']), _row(P['st_lw_b0']),
                      P['st_lw_w1'], _row(P['st_lw_b1']))

    st, ss, ss2 = _agg(h1, hs, hs2, _row(P['st_lw_g1']), _row(P['st_lw_bb1']),
                       P['st_lw_w2'], _row(P['st_lw_b2']),
                       kvg, pg, prep, lpw1, lpb1, prs, prs2,
                       lpg, lpbb, lpw2, lpb2)

    pc_cat, kc_cat = _tail(
        st, k, ss, ss2, _row(P['bn2_g']), _row(P['bn2_b']),
        P['ps_w1'], _row(P['ps_b1']), _row(P['ps_g']), _row(P['ps_bb']),
        P['ps_w2'], _row(P['ps_b2']),
        P['psconv_w'][:, :, 0], P['psconv_w'][:, :, 1],
        P['mdf_w1'], _row(P['mdf_b1']), _row(P['mdf_g']), _row(P['mdf_bb']),
        P['mdf_w2'], _row(P['mdf_b2']),
        P['sc_w'], _row(P['sc_b']), _row(P['sc_g']), _row(P['sc_bb']),
        P['md_w1'], _row(P['md_b1']), _row(P['md_g']), _row(P['md_bb']),
        P['md_w2'], _row(P['md_b2']), pcd_prev)

    pcd_child = pc_cat.reshape(2, _N, 3).transpose(1, 0, 2).reshape(2 * _N, 3)
    k_curr = kc_cat.reshape(2, _N, 128).transpose(1, 0, 2).reshape(2 * _N, 128)
    return pcd_child, k_curr


# speedup vs baseline: 5.4366x; 1.0459x over previous
import functools

import jax
import jax.numpy as jnp
from jax import lax
from jax.experimental import pallas as pl
from jax.experimental.pallas import tpu as pltpu
from jax.experimental.pallas import tpu_sc as plsc

_N = 10000
_SEG = 5000
_SEGP = 5120
_NS = 16
_ROWS = _N * _NS
_EPS = 1e-5
_KNN_R = 256
_PB = 400
_RB = _PB * _NS
_GB = _N // _PB
_INTERPRET = False


def _f32dot(a, b):
    return jnp.dot(a, b, preferred_element_type=jnp.float32)


def _front_body(kp, l1w, l1b, g1, b1, wq, bq, wk, bk, wv, bv,
                ko, xqo, xko, xvo):
    t = _f32dot(kp[...], l1w[...]) + l1b[...]
    m = jnp.mean(t, axis=0, keepdims=True)
    v = jnp.mean(jnp.square(t - m), axis=0, keepdims=True)
    k = jnp.maximum((t - m) / jnp.sqrt(v + _EPS) * g1[...] + b1[...], 0.0)
    ko[...] = k
    xqo[...] = _f32dot(k, wq[...]) + bq[...]
    xko[...] = _f32dot(k, wk[...]) + bk[...]
    xvo[...] = _f32dot(k, wv[...]) + bv[...]


def _front(kprev, l1w, l1b, g1, b1, wq, bq, wk, bk, wv, bv):
    o = jax.ShapeDtypeStruct((_N, 128), jnp.float32)
    return pl.pallas_call(
        _front_body,
        out_shape=[o, o, o, o],
        interpret=_INTERPRET,
    )(kprev, l1w, l1b, g1, b1, wq, bq, wk, bk, wv, bv)


def _knn_body(prow, pall, idxo):
    pr = prow[...]
    pa = pall[...]
    dots = lax.dot_general(pr, pa, (((1,), (1,)), ((), ())),
                           preferred_element_type=jnp.float32)
    sqa = jnp.sum(pa * pa, axis=1, keepdims=True)
    d = jnp.transpose(sqa) - 2.0 * dots
    col = lax.broadcasted_iota(jnp.int32, d.shape, 1)
    inf = jnp.float32(jnp.inf)
    d = jnp.where(col >= _SEG, inf, d)
    outs = []
    for _ in range(_NS):
        mval = jnp.min(d, axis=1, keepdims=True)
        cand = jnp.where(d <= mval, col, jnp.int32(1 << 30))
        sel = jnp.min(cand, axis=1, keepdims=True)
        outs.append(sel)
        d = jnp.where(col == sel, inf, d)
    idxo[...] = jnp.concatenate(outs, axis=1)


def _knn_segment(p_seg_padded):
    return pl.pallas_call(
        _knn_body,
        grid=(_SEGP // _KNN_R,),
        in_specs=[
            pl.BlockSpec((_KNN_R, 3), lambda i: (i, 0)),
            pl.BlockSpec((_SEGP, 3), lambda i: (0, 0)),
        ],
        out_specs=pl.BlockSpec((_KNN_R, _NS), lambda i: (i, 0)),
        out_shape=jax.ShapeDtypeStruct((_SEGP, _NS), jnp.int32),
        interpret=_INTERPRET,
    )(p_seg_padded, p_seg_padded)[:_SEG]


def _gather2(kv, pp, idx2):
    nchunks = idx2.shape[0]
    mesh = plsc.VectorSubcoreMesh(core_axis_name="c", subcore_axis_name="s")

    @functools.partial(
        pl.kernel,
        mesh=mesh,
        out_type=[
            jax.ShapeDtypeStruct((_ROWS, 256), jnp.float32),
            jax.ShapeDtypeStruct((_ROWS, 128), jnp.float32),
        ],
        scratch_types=[
            pltpu.VMEM((128,), jnp.int32),
            pltpu.VMEM((128, 256), jnp.float32),
            pltpu.VMEM((128, 128), jnp.float32),
            pltpu.SemaphoreType.DMA,
            pltpu.SemaphoreType.DMA,
        ],
    )
    def gk(kv_hbm, pp_hbm, idx_hbm, okv_hbm, op_hbm,
           idx_v, kvb, pb, sem, sem2):
        wid = lax.axis_index("s") * 2 + lax.axis_index("c")
        extra = jnp.where(wid < nchunks % 32, 1, 0)
        nj = nchunks // 32 + extra

        def body(j, carry):
            c = wid + 32 * j
            pltpu.sync_copy(idx_hbm.at[c], idx_v)
            ck = pltpu.async_copy(kv_hbm.at[idx_v], kvb, sem)
            cp = pltpu.async_copy(pp_hbm.at[idx_v], pb, sem2)
            ck.wait()
            pltpu.sync_copy(kvb, okv_hbm.at[pl.ds(c * 128, 128)])
            cp.wait()
            pltpu.sync_copy(pb, op_hbm.at[pl.ds(c * 128, 128)])
            return carry

        lax.fori_loop(0, nj, body, 0)

    return gk(kv, pp, idx2)


def _prstats_body(pg, prep, w1, b1, so, s2o):
    gx = pg[...] - prep[...]
    t = _f32dot(gx, w1[...]) + b1[...]

    @pl.when(pl.program_id(0) == 0)
    def _():
        so[...] = jnp.zeros_like(so)
        s2o[...] = jnp.zeros_like(s2o)

    so[...] += jnp.sum(t, axis=0, keepdims=True)
    s2o[...] += jnp.sum(t * t, axis=0, keepdims=True)


def _prstats(pg, prep, w1, b1):
    row = lambda i: (i, 0)
    fix = lambda i: (0, 0)
    o = jax.ShapeDtypeStruct((1, 8), jnp.float32)
    return pl.pallas_call(
        _prstats_body,
        grid=(_GB,),
        in_specs=[
            pl.BlockSpec((_RB, 8), row),
            pl.BlockSpec((_RB, 8), row),
            pl.BlockSpec((8, 8), fix),
            pl.BlockSpec((1, 8), fix),
        ],
        out_specs=[pl.BlockSpec((1, 8), fix), pl.BlockSpec((1, 8), fix)],
        out_shape=[o, o],
        interpret=_INTERPRET,
    )(pg, prep, w1, b1)


def _pr_block(pg, prep, w1, b1, prs, prs2, g, bb, w2, b2):
    gx = pg - prep
    t = _f32dot(gx, w1) + b1
    m = prs / jnp.float32(_ROWS)
    v = prs2 / jnp.float32(_ROWS) - m * m
    tn = jnp.maximum((t - m) / jnp.sqrt(v + _EPS) * g + bb, 0.0)
    return _f32dot(tn, w2) + b2


def _wpre_body(xkg, xq, pg, prep, w1, b1, prs, prs2, g, bb, w2, b2,
               wpo, so, s2o):
    p_r = _pr_block(pg[...], prep[...], w1[...], b1[...], prs[...],
                    prs2[...], g[...], bb[...], w2[...], b2[...])
    xqe = jnp.broadcast_to(xq[...][:, None, :], (_PB, _NS, 128)).reshape(_RB, 128)
    w = xkg[...] - xqe + p_r
    wpo[...] = w

    @pl.when(pl.program_id(0) == 0)
    def _():
        so[...] = jnp.zeros_like(so)
        s2o[...] = jnp.zeros_like(s2o)

    so[...] += jnp.sum(w, axis=0, keepdims=True)
    s2o[...] += jnp.sum(w * w, axis=0, keepdims=True)


def _wpre(xkg, xq, pg, prep, w1, b1, prs, prs2, g, bb, w2, b2):
    row = lambda i: (i, 0)
    fix = lambda i: (0, 0)
    return pl.pallas_call(
        _wpre_body,
        grid=(_GB,),
        in_specs=[
            pl.BlockSpec((_RB, 128), row),
            pl.BlockSpec((_PB, 128), row),
            pl.BlockSpec((_RB, 8), row),
            pl.BlockSpec((_RB, 8), row),
            pl.BlockSpec((8, 8), fix),
            pl.BlockSpec((1, 8), fix),
            pl.BlockSpec((1, 8), fix),
            pl.BlockSpec((1, 8), fix),
            pl.BlockSpec((1, 8), fix),
            pl.BlockSpec((1, 8), fix),
            pl.BlockSpec((8, 128), fix),
            pl.BlockSpec((1, 128), fix),
        ],
        out_specs=[
            pl.BlockSpec((_RB, 128), row),
            pl.BlockSpec((1, 128), fix),
            pl.BlockSpec((1, 128), fix),
        ],
        out_shape=[
            jax.ShapeDtypeStruct((_ROWS, 128), jnp.float32),
            jax.ShapeDtypeStruct((1, 128), jnp.float32),
            jax.ShapeDtypeStruct((1, 128), jnp.float32),
        ],
        interpret=_INTERPRET,
    )(xkg, xq, pg, prep, w1, b1, prs, prs2, g, bb, w2, b2)


def _h1_body(wp, ws, ws2, g0, b0, w1, b1, h1o, so, s2o):
    m = ws[...] / jnp.float32(_ROWS)
    v = ws2[...] / jnp.float32(_ROWS) - m * m
    wn = jnp.maximum((wp[...] - m) / jnp.sqrt(v + _EPS) * g0[...] + b0[...], 0.0)
    h1 = _f32dot(wn, w1[...]) + b1[...]
    h1o[...] = h1

    @pl.when(pl.program_id(0) == 0)
    def _():
        so[...] = jnp.zeros_like(so)
        s2o[...] = jnp.zeros_like(s2o)

    so[...] += jnp.sum(h1, axis=0, keepdims=True)
    s2o[...] += jnp.sum(h1 * h1, axis=0, keepdims=True)


def _h1(wp, ws, ws2, g0, b0, w1, b1):
    row = lambda i: (i, 0)
    fix = lambda i: (0, 0)
    return pl.pallas_call(
        _h1_body,
        grid=(_GB,),
        in_specs=[
            pl.BlockSpec((_RB, 128), row),
            pl.BlockSpec((1, 128), fix),
            pl.BlockSpec((1, 128), fix),
            pl.BlockSpec((1, 128), fix),
            pl.BlockSpec((1, 128), fix),
            pl.BlockSpec((128, 16), fix),
            pl.BlockSpec((1, 16), fix),
        ],
        out_specs=[
            pl.BlockSpec((_RB, 16), row),
            pl.BlockSpec((1, 16), fix),
            pl.BlockSpec((1, 16), fix),
        ],
        out_shape=[
            jax.ShapeDtypeStruct((_ROWS, 16), jnp.float32),
            jax.ShapeDtypeStruct((1, 16), jnp.float32),
            jax.ShapeDtypeStruct((1, 16), jnp.float32),
        ],
        interpret=_INTERPRET,
    )(wp, ws, ws2, g0, b0, w1, b1)


def _agg_body(h1, hs, hs2, g1, bb1, w2, b2,
              gv, pg, prep, lpw1, lpb1, prs, prs2, lpg, lpbb, lpw2, lpb2,
              oo, so, s2o):
    m = hs[...] / jnp.float32(_ROWS)
    v = hs2[...] / jnp.float32(_ROWS) - m * m
    h = jnp.maximum((h1[...] - m) / jnp.sqrt(v + _EPS) * g1[...] + bb1[...], 0.0)
    w = _f32dot(h, w2[...]) + b2[...]
    w3 = w.reshape(_PB, _NS, _NS)
    mx = jnp.max(w3, axis=1, keepdims=True)
    e = jnp.exp(w3 - mx)
    sm = e / jnp.sum(e, axis=1, keepdims=True)
    p_r = _pr_block(pg[...], prep[...], lpw1[...], lpb1[...], prs[...],
                    prs2[...], lpg[...], lpbb[...], lpw2[...], lpb2[...])
    val = (gv[...] + p_r).reshape(_PB, _NS, 128)
    wt = jnp.concatenate([sm] * 8, axis=2)
    ob = jnp.sum(val * wt, axis=1)
    oo[...] = ob

    @pl.when(pl.program_id(0) == 0)
    def _():
        so[...] = jnp.zeros_like(so)
        s2o[...] = jnp.zeros_like(s2o)

    so[...] += jnp.sum(ob, axis=0, keepdims=True)
    s2o[...] += jnp.sum(ob * ob, axis=0, keepdims=True)


def _agg(h1, hs, hs2, g1, bb1, w2, b2,
         gv, pg, prep, lpw1, lpb1, prs, prs2, lpg, lpbb, lpw2, lpb2):
    row = lambda i: (i, 0)
    fix = lambda i: (0, 0)
    return pl.pallas_call(
        _agg_body,
        grid=(_GB,),
        in_specs=[
            pl.BlockSpec((_RB, 16), row),
            pl.BlockSpec((1, 16), fix),
            pl.BlockSpec((1, 16), fix),
            pl.BlockSpec((1, 16), fix),
            pl.BlockSpec((1, 16), fix),
            pl.BlockSpec((16, 16), fix),
            pl.BlockSpec((1, 16), fix),
            pl.BlockSpec((_RB, 128), lambda i: (i, 1)),
            pl.BlockSpec((_RB, 8), row),
            pl.BlockSpec((_RB, 8), row),
            pl.BlockSpec((8, 8), fix),
            pl.BlockSpec((1, 8), fix),
            pl.BlockSpec((1, 8), fix),
            pl.BlockSpec((1, 8), fix),
            pl.BlockSpec((1, 8), fix),
            pl.BlockSpec((1, 8), fix),
            pl.BlockSpec((8, 128), fix),
            pl.BlockSpec((1, 128), fix),
        ],
        out_specs=[
            pl.BlockSpec((_PB, 128), row),
            pl.BlockSpec((1, 128), fix),
            pl.BlockSpec((1, 128), fix),
        ],
        out_shape=[
            jax.ShapeDtypeStruct((_N, 128), jnp.float32),
            jax.ShapeDtypeStruct((1, 128), jnp.float32),
            jax.ShapeDtypeStruct((1, 128), jnp.float32),
        ],
        interpret=_INTERPRET,
    )(h1, hs, hs2, g1, bb1, w2, b2,
      gv, pg, prep, lpw1, lpb1, prs, prs2, lpg, lpbb, lpw2, lpb2)


def _bn_rows(x, g, b):
    m = jnp.mean(x, axis=0, keepdims=True)
    v = jnp.mean(jnp.square(x - m), axis=0, keepdims=True)
    return (x - m) / jnp.sqrt(v + _EPS) * g + b


def _tail_body(st, kk, s2, s2q, g2, b2, psw1, psb1, psg, psbb, psw2, psb2,
               cw0, cw1, mdfw1, mdfb1, mdfg, mdfbb, mdfw2, mdfb2,
               scw, scb, scg, scbb, mdw1, mdb1, mdg, mdbb, mdw2, mdb2,
               pcd, pco, kco):
    m = s2[...] / jnp.float32(_N)
    v = s2q[...] / jnp.float32(_N) - m * m
    h = jnp.maximum((st[...] - m) / jnp.sqrt(v + _EPS) * g2[...] + b2[...], 0.0)
    h = h + kk[...]
    t1 = jnp.maximum(_bn_rows(_f32dot(h, psw1[...]) + psb1[...],
                              psg[...], psbb[...]), 0.0)
    fc = _f32dot(t1, psw2[...]) + psb2[...]
    a0 = _f32dot(fc, cw0[...])
    a1 = _f32dot(fc, cw1[...])
    feat = jnp.concatenate([a0, a1], axis=0)
    kc = jnp.maximum(_bn_rows(_f32dot(feat, mdfw1[...]) + mdfb1[...],
                              mdfg[...], mdfbb[...]), 0.0)
    kc = _f32dot(kc, mdfw2[...]) + mdfb2[...]
    sc_ = _bn_rows(_f32dot(feat, scw[...]) + scb[...], scg[...], scbb[...])
    k_curr = jnp.maximum(kc + sc_, 0.0)
    md = jnp.maximum(_bn_rows(_f32dot(k_curr, mdw1[...]) + mdb1[...],
                              mdg[...], mdbb[...]), 0.0)
    delta = jnp.tanh(jnp.tanh(_f32dot(md, mdw2[...]) + mdb2[...]))
    p = pcd[...]
    pco[...] = jnp.concatenate([p, p], axis=0) + delta
    kco[...] = k_curr


def _tail(st, kk, s2, s2q, g2, b2, psw1, psb1, psg, psbb, psw2, psb2,
          cw0, cw1, mdfw1, mdfb1, mdfg, mdfbb, mdfw2, mdfb2,
          scw, scb, scg, scbb, mdw1, mdb1, mdg, mdbb, mdw2, mdb2, pcd):
    return pl.pallas_call(
        _tail_body,
        out_shape=[
            jax.ShapeDtypeStruct((2 * _N, 3), jnp.float32),
            jax.ShapeDtypeStruct((2 * _N, 128), jnp.float32),
        ],
        interpret=_INTERPRET,
    )(st, kk, s2, s2q, g2, b2, psw1, psb1, psg, psbb, psw2, psb2,
      cw0, cw1, mdfw1, mdfb1, mdfg, mdfbb, mdfw2, mdfb2,
      scw, scb, scg, scbb, mdw1, mdb1, mdg, mdbb, mdw2, mdb2, pcd)


def _row(x):
    return jnp.asarray(x, jnp.float32).reshape(1, -1)


def _pad_lane(x, n):
    return jnp.pad(jnp.asarray(x, jnp.float32), ((0, 0), (0, n - x.shape[1])))


def kernel(pcd_prev, K_prev, e_p, e_x, e_o, params):
    P = params
    del e_p, e_x, e_o

    k, xq, xk, xv = _front(
        K_prev, P['l1_w'], _row(P['l1_b']), _row(P['bn1_g']), _row(P['bn1_b']),
        P['st_lq_w'], _row(P['st_lq_b']), P['st_lk_w'], _row(P['st_lk_b']),
        P['st_lv_w'], _row(P['st_lv_b']))

    p0 = jnp.pad(pcd_prev[:_SEG], ((0, _SEGP - _SEG), (0, 0)))
    p1 = jnp.pad(pcd_prev[_SEG:], ((0, _SEGP - _SEG), (0, 0)))
    idx0 = _knn_segment(p0)
    idx1 = _knn_segment(p1) + _SEG
    idx2 = jnp.concatenate([idx0, idx1], axis=0).reshape(_ROWS // 128, 128)

    kv = jnp.concatenate([xk, xv], axis=1)
    pp128 = jnp.pad(pcd_prev, ((0, 0), (0, 125)))
    kvg, pgf = _gather2(kv, pp128, idx2)
    pg = pgf[:, :8]

    prep = jnp.repeat(pp128[:, :8], _NS, axis=0)

    lpw1 = jnp.zeros((8, 8), jnp.float32).at[:3, :3].set(P['st_lp_w1'])
    lpb1 = jnp.zeros((1, 8), jnp.float32).at[:, :3].set(P['st_lp_b1'])
    lpg = jnp.zeros((1, 8), jnp.float32).at[:, :3].set(P['st_lp_g'])
    lpbb = jnp.zeros((1, 8), jnp.float32).at[:, :3].set(P['st_lp_bb'])
    lpw2 = jnp.zeros((8, 128), jnp.float32).at[:3, :].set(P['st_lp_w2'])
    lpb2 = _row(P['st_lp_b2'])

    prs, prs2 = _prstats(pg, prep, lpw1, lpb1)

    wp, ws, ws2 = _wpre(kvg, xq, pg, prep, lpw1, lpb1, prs, prs2,
                        lpg, lpbb, lpw2, lpb2)

    h1, hs, hs2 = _h1(wp, ws, ws2, _row(P['st_lw_g0']), _row(P['st_lw_b0']),
                      P['st_lw_w1'], _row(P['st_lw_b1']))

    st, ss, ss2 = _agg(h1, hs, hs2, _row(P['st_lw_g1']), _row(P['st_lw_bb1']),
                       P['st_lw_w2'], _row(P['st_lw_b2']),
                       kvg, pg, prep, lpw1, lpb1, prs, prs2,
                       lpg, lpbb, lpw2, lpb2)

    pc_cat, kc_cat = _tail(
        st, k, ss, ss2, _row(P['bn2_g']), _row(P['bn2_b']),
        P['ps_w1'], _row(P['ps_b1']), _row(P['ps_g']), _row(P['ps_bb']),
        P['ps_w2'], _row(P['ps_b2']),
        P['psconv_w'][:, :, 0], P['psconv_w'][:, :, 1],
        P['mdf_w1'], _row(P['mdf_b1']), _row(P['mdf_g']), _row(P['mdf_bb']),
        P['mdf_w2'], _row(P['mdf_b2']),
        P['sc_w'], _row(P['sc_b']), _row(P['sc_g']), _row(P['sc_bb']),
        P['md_w1'], _row(P['md_b1']), _row(P['md_g']), _row(P['md_bb']),
        P['md_w2'], _row(P['md_b2']), pcd_prev)

    pcd_child = pc_cat.reshape(2, _N, 3).transpose(1, 0, 2).reshape(2 * _N, 3)
    k_curr = kc_cat.reshape(2, _N, 128).transpose(1, 0, 2).reshape(2 * _N, 128)
    return pcd_child, k_curr


# speedup vs baseline: 6.0304x; 1.1092x over previous
import functools

import jax
import jax.numpy as jnp
from jax import lax
from jax.experimental import pallas as pl
from jax.experimental.pallas import tpu as pltpu
from jax.experimental.pallas import tpu_sc as plsc

_N = 10000
_SEG = 5000
_SEGP = 5120
_NS = 16
_ROWS = _N * _NS
_EPS = 1e-5
_KNN_R = 256
_PB = 400
_RB = _PB * _NS
_GB = _N // _PB
_INTERPRET = False


def _f32dot(a, b):
    return jnp.dot(a, b, preferred_element_type=jnp.float32)


def _front_body(kp, l1w, l1b, g1, b1, wq, bq, wk, bk, wv, bv,
                ko, xqo, xko, xvo):
    t = _f32dot(kp[...], l1w[...]) + l1b[...]
    m = jnp.mean(t, axis=0, keepdims=True)
    v = jnp.mean(jnp.square(t - m), axis=0, keepdims=True)
    k = jnp.maximum((t - m) / jnp.sqrt(v + _EPS) * g1[...] + b1[...], 0.0)
    ko[...] = k
    xqo[...] = _f32dot(k, wq[...]) + bq[...]
    xko[...] = _f32dot(k, wk[...]) + bk[...]
    xvo[...] = _f32dot(k, wv[...]) + bv[...]


def _front(kprev, l1w, l1b, g1, b1, wq, bq, wk, bk, wv, bv):
    o = jax.ShapeDtypeStruct((_N, 128), jnp.float32)
    return pl.pallas_call(
        _front_body,
        out_shape=[o, o, o, o],
        interpret=_INTERPRET,
    )(kprev, l1w, l1b, g1, b1, wq, bq, wk, bk, wv, bv)


def _knn_body(prow, pall, idxo):
    pr = prow[...]
    pa = pall[...]
    dots = lax.dot_general(pr, pa, (((1,), (1,)), ((), ())),
                           preferred_element_type=jnp.float32)
    sqa = jnp.sum(pa * pa, axis=1, keepdims=True)
    d = jnp.transpose(sqa) - 2.0 * dots
    colf = lax.broadcasted_iota(jnp.int32, d.shape, 1).astype(jnp.float32)
    inf = jnp.float32(jnp.inf)
    d = jnp.where(colf >= _SEG, inf, d)
    outs = []
    mval = jnp.min(d, axis=1, keepdims=True)
    for it in range(_NS):
        cand = jnp.where(d <= mval, colf, inf)
        sel = jnp.min(cand, axis=1, keepdims=True)
        outs.append(sel)
        if it + 1 < _NS:
            d = jnp.where(colf == sel, inf, d)
            mval = jnp.min(d, axis=1, keepdims=True)
    idxo[...] = jnp.concatenate(outs, axis=1).astype(jnp.int32)


def _knn_segment(p_seg_padded):
    return pl.pallas_call(
        _knn_body,
        grid=(_SEGP // _KNN_R,),
        in_specs=[
            pl.BlockSpec((_KNN_R, 3), lambda i: (i, 0)),
            pl.BlockSpec((_SEGP, 3), lambda i: (0, 0)),
        ],
        out_specs=pl.BlockSpec((_KNN_R, _NS), lambda i: (i, 0)),
        out_shape=jax.ShapeDtypeStruct((_SEGP, _NS), jnp.int32),
        interpret=_INTERPRET,
    )(p_seg_padded, p_seg_padded)[:_SEG]


def _gather2(kv, pp, idx2):
    nchunks = idx2.shape[0]
    mesh = plsc.VectorSubcoreMesh(core_axis_name="c", subcore_axis_name="s")

    @functools.partial(
        pl.kernel,
        mesh=mesh,
        out_type=[
            jax.ShapeDtypeStruct((_ROWS, 256), jnp.float32),
            jax.ShapeDtypeStruct((_ROWS, 128), jnp.float32),
        ],
        scratch_types=[
            pltpu.VMEM((128,), jnp.int32),
            pltpu.VMEM((128, 256), jnp.float32),
            pltpu.VMEM((128, 128), jnp.float32),
            pltpu.SemaphoreType.DMA,
            pltpu.SemaphoreType.DMA,
        ],
    )
    def gk(kv_hbm, pp_hbm, idx_hbm, okv_hbm, op_hbm,
           idx_v, kvb, pb, sem, sem2):
        wid = lax.axis_index("s") * 2 + lax.axis_index("c")
        extra = jnp.where(wid < nchunks % 32, 1, 0)
        nj = nchunks // 32 + extra

        def body(j, carry):
            c = wid + 32 * j
            pltpu.sync_copy(idx_hbm.at[c], idx_v)
            ck = pltpu.async_copy(kv_hbm.at[idx_v], kvb, sem)
            cp = pltpu.async_copy(pp_hbm.at[idx_v], pb, sem2)
            ck.wait()
            pltpu.sync_copy(kvb, okv_hbm.at[pl.ds(c * 128, 128)])
            cp.wait()
            pltpu.sync_copy(pb, op_hbm.at[pl.ds(c * 128, 128)])
            return carry

        lax.fori_loop(0, nj, body, 0)

    return gk(kv, pp, idx2)


def _prstats_body(pg, prep, w1, b1, so, s2o):
    gx = pg[...] - prep[...]
    t = _f32dot(gx, w1[...]) + b1[...]

    @pl.when(pl.program_id(0) == 0)
    def _():
        so[...] = jnp.zeros_like(so)
        s2o[...] = jnp.zeros_like(s2o)

    so[...] += jnp.sum(t, axis=0, keepdims=True)
    s2o[...] += jnp.sum(t * t, axis=0, keepdims=True)


def _prstats(pg, prep, w1, b1):
    row = lambda i: (i, 0)
    fix = lambda i: (0, 0)
    o = jax.ShapeDtypeStruct((1, 8), jnp.float32)
    return pl.pallas_call(
        _prstats_body,
        grid=(_GB,),
        in_specs=[
            pl.BlockSpec((_RB, 8), row),
            pl.BlockSpec((_RB, 8), row),
            pl.BlockSpec((8, 8), fix),
            pl.BlockSpec((1, 8), fix),
        ],
        out_specs=[pl.BlockSpec((1, 8), fix), pl.BlockSpec((1, 8), fix)],
        out_shape=[o, o],
        interpret=_INTERPRET,
    )(pg, prep, w1, b1)


def _pr_block(pg, prep, w1, b1, prs, prs2, g, bb, w2, b2):
    gx = pg - prep
    t = _f32dot(gx, w1) + b1
    m = prs / jnp.float32(_ROWS)
    v = prs2 / jnp.float32(_ROWS) - m * m
    tn = jnp.maximum((t - m) / jnp.sqrt(v + _EPS) * g + bb, 0.0)
    return _f32dot(tn, w2) + b2


def _wpre_body(xkg, xq, pg, prep, w1, b1, prs, prs2, g, bb, w2, b2,
               wpo, so, s2o):
    p_r = _pr_block(pg[...], prep[...], w1[...], b1[...], prs[...],
                    prs2[...], g[...], bb[...], w2[...], b2[...])
    xqe = jnp.broadcast_to(xq[...][:, None, :], (_PB, _NS, 128)).reshape(_RB, 128)
    w = xkg[...] - xqe + p_r
    wpo[...] = w

    @pl.when(pl.program_id(0) == 0)
    def _():
        so[...] = jnp.zeros_like(so)
        s2o[...] = jnp.zeros_like(s2o)

    so[...] += jnp.sum(w, axis=0, keepdims=True)
    s2o[...] += jnp.sum(w * w, axis=0, keepdims=True)


def _wpre(xkg, xq, pg, prep, w1, b1, prs, prs2, g, bb, w2, b2):
    row = lambda i: (i, 0)
    fix = lambda i: (0, 0)
    return pl.pallas_call(
        _wpre_body,
        grid=(_GB,),
        in_specs=[
            pl.BlockSpec((_RB, 128), row),
            pl.BlockSpec((_PB, 128), row),
            pl.BlockSpec((_RB, 8), row),
            pl.BlockSpec((_RB, 8), row),
            pl.BlockSpec((8, 8), fix),
            pl.BlockSpec((1, 8), fix),
            pl.BlockSpec((1, 8), fix),
            pl.BlockSpec((1, 8), fix),
            pl.BlockSpec((1, 8), fix),
            pl.BlockSpec((1, 8), fix),
            pl.BlockSpec((8, 128), fix),
            pl.BlockSpec((1, 128), fix),
        ],
        out_specs=[
            pl.BlockSpec((_RB, 128), row),
            pl.BlockSpec((1, 128), fix),
            pl.BlockSpec((1, 128), fix),
        ],
        out_shape=[
            jax.ShapeDtypeStruct((_ROWS, 128), jnp.float32),
            jax.ShapeDtypeStruct((1, 128), jnp.float32),
            jax.ShapeDtypeStruct((1, 128), jnp.float32),
        ],
        interpret=_INTERPRET,
    )(xkg, xq, pg, prep, w1, b1, prs, prs2, g, bb, w2, b2)


def _h1_body(wp, ws, ws2, g0, b0, w1, b1, h1o, so, s2o):
    m = ws[...] / jnp.float32(_ROWS)
    v = ws2[...] / jnp.float32(_ROWS) - m * m
    wn = jnp.maximum((wp[...] - m) / jnp.sqrt(v + _EPS) * g0[...] + b0[...], 0.0)
    h1 = _f32dot(wn, w1[...]) + b1[...]
    h1o[...] = h1

    @pl.when(pl.program_id(0) == 0)
    def _():
        so[...] = jnp.zeros_like(so)
        s2o[...] = jnp.zeros_like(s2o)

    so[...] += jnp.sum(h1, axis=0, keepdims=True)
    s2o[...] += jnp.sum(h1 * h1, axis=0, keepdims=True)


def _h1(wp, ws, ws2, g0, b0, w1, b1):
    row = lambda i: (i, 0)
    fix = lambda i: (0, 0)
    return pl.pallas_call(
        _h1_body,
        grid=(_GB,),
        in_specs=[
            pl.BlockSpec((_RB, 128), row),
            pl.BlockSpec((1, 128), fix),
            pl.BlockSpec((1, 128), fix),
            pl.BlockSpec((1, 128), fix),
            pl.BlockSpec((1, 128), fix),
            pl.BlockSpec((128, 16), fix),
            pl.BlockSpec((1, 16), fix),
        ],
        out_specs=[
            pl.BlockSpec((_RB, 16), row),
            pl.BlockSpec((1, 16), fix),
            pl.BlockSpec((1, 16), fix),
        ],
        out_shape=[
            jax.ShapeDtypeStruct((_ROWS, 16), jnp.float32),
            jax.ShapeDtypeStruct((1, 16), jnp.float32),
            jax.ShapeDtypeStruct((1, 16), jnp.float32),
        ],
        interpret=_INTERPRET,
    )(wp, ws, ws2, g0, b0, w1, b1)


def _agg_body(h1, hs, hs2, g1, bb1, w2, b2,
              gv, pg, prep, lpw1, lpb1, prs, prs2, lpg, lpbb, lpw2, lpb2,
              oo, so, s2o):
    m = hs[...] / jnp.float32(_ROWS)
    v = hs2[...] / jnp.float32(_ROWS) - m * m
    h = jnp.maximum((h1[...] - m) / jnp.sqrt(v + _EPS) * g1[...] + bb1[...], 0.0)
    w = _f32dot(h, w2[...]) + b2[...]
    w3 = w.reshape(_PB, _NS, _NS)
    mx = jnp.max(w3, axis=1, keepdims=True)
    e = jnp.exp(w3 - mx)
    sm = e / jnp.sum(e, axis=1, keepdims=True)
    p_r = _pr_block(pg[...], prep[...], lpw1[...], lpb1[...], prs[...],
                    prs2[...], lpg[...], lpbb[...], lpw2[...], lpb2[...])
    val = (gv[...] + p_r).reshape(_PB, _NS, 128)
    wt = jnp.concatenate([sm] * 8, axis=2)
    ob = jnp.sum(val * wt, axis=1)
    oo[...] = ob

    @pl.when(pl.program_id(0) == 0)
    def _():
        so[...] = jnp.zeros_like(so)
        s2o[...] = jnp.zeros_like(s2o)

    so[...] += jnp.sum(ob, axis=0, keepdims=True)
    s2o[...] += jnp.sum(ob * ob, axis=0, keepdims=True)


def _agg(h1, hs, hs2, g1, bb1, w2, b2,
         gv, pg, prep, lpw1, lpb1, prs, prs2, lpg, lpbb, lpw2, lpb2):
    row = lambda i: (i, 0)
    fix = lambda i: (0, 0)
    return pl.pallas_call(
        _agg_body,
        grid=(_GB,),
        in_specs=[
            pl.BlockSpec((_RB, 16), row),
            pl.BlockSpec((1, 16), fix),
            pl.BlockSpec((1, 16), fix),
            pl.BlockSpec((1, 16), fix),
            pl.BlockSpec((1, 16), fix),
            pl.BlockSpec((16, 16), fix),
            pl.BlockSpec((1, 16), fix),
            pl.BlockSpec((_RB, 128), lambda i: (i, 1)),
            pl.BlockSpec((_RB, 8), row),
            pl.BlockSpec((_RB, 8), row),
            pl.BlockSpec((8, 8), fix),
            pl.BlockSpec((1, 8), fix),
            pl.BlockSpec((1, 8), fix),
            pl.BlockSpec((1, 8), fix),
            pl.BlockSpec((1, 8), fix),
            pl.BlockSpec((1, 8), fix),
            pl.BlockSpec((8, 128), fix),
            pl.BlockSpec((1, 128), fix),
        ],
        out_specs=[
            pl.BlockSpec((_PB, 128), row),
            pl.BlockSpec((1, 128), fix),
            pl.BlockSpec((1, 128), fix),
        ],
        out_shape=[
            jax.ShapeDtypeStruct((_N, 128), jnp.float32),
            jax.ShapeDtypeStruct((1, 128), jnp.float32),
            jax.ShapeDtypeStruct((1, 128), jnp.float32),
        ],
        interpret=_INTERPRET,
    )(h1, hs, hs2, g1, bb1, w2, b2,
      gv, pg, prep, lpw1, lpb1, prs, prs2, lpg, lpbb, lpw2, lpb2)


def _bn_rows(x, g, b):
    m = jnp.mean(x, axis=0, keepdims=True)
    v = jnp.mean(jnp.square(x - m), axis=0, keepdims=True)
    return (x - m) / jnp.sqrt(v + _EPS) * g + b


def _tail_body(st, kk, s2, s2q, g2, b2, psw1, psb1, psg, psbb, psw2, psb2,
               cw0, cw1, mdfw1, mdfb1, mdfg, mdfbb, mdfw2, mdfb2,
               scw, scb, scg, scbb, mdw1, mdb1, mdg, mdbb, mdw2, mdb2,
               pcd, pco, kco):
    m = s2[...] / jnp.float32(_N)
    v = s2q[...] / jnp.float32(_N) - m * m
    h = jnp.maximum((st[...] - m) / jnp.sqrt(v + _EPS) * g2[...] + b2[...], 0.0)
    h = h + kk[...]
    t1 = jnp.maximum(_bn_rows(_f32dot(h, psw1[...]) + psb1[...],
                              psg[...], psbb[...]), 0.0)
    fc = _f32dot(t1, psw2[...]) + psb2[...]
    a0 = _f32dot(fc, cw0[...])
    a1 = _f32dot(fc, cw1[...])
    feat = jnp.concatenate([a0, a1], axis=0)
    kc = jnp.maximum(_bn_rows(_f32dot(feat, mdfw1[...]) + mdfb1[...],
                              mdfg[...], mdfbb[...]), 0.0)
    kc = _f32dot(kc, mdfw2[...]) + mdfb2[...]
    sc_ = _bn_rows(_f32dot(feat, scw[...]) + scb[...], scg[...], scbb[...])
    k_curr = jnp.maximum(kc + sc_, 0.0)
    md = jnp.maximum(_bn_rows(_f32dot(k_curr, mdw1[...]) + mdb1[...],
                              mdg[...], mdbb[...]), 0.0)
    delta = jnp.tanh(jnp.tanh(_f32dot(md, mdw2[...]) + mdb2[...]))
    p = pcd[...]
    pco[...] = jnp.concatenate([p, p], axis=0) + delta
    kco[...] = k_curr


def _tail(st, kk, s2, s2q, g2, b2, psw1, psb1, psg, psbb, psw2, psb2,
          cw0, cw1, mdfw1, mdfb1, mdfg, mdfbb, mdfw2, mdfb2,
          scw, scb, scg, scbb, mdw1, mdb1, mdg, mdbb, mdw2, mdb2, pcd):
    return pl.pallas_call(
        _tail_body,
        out_shape=[
            jax.ShapeDtypeStruct((2 * _N, 3), jnp.float32),
            jax.ShapeDtypeStruct((2 * _N, 128), jnp.float32),
        ],
        interpret=_INTERPRET,
    )(st, kk, s2, s2q, g2, b2, psw1, psb1, psg, psbb, psw2, psb2,
      cw0, cw1, mdfw1, mdfb1, mdfg, mdfbb, mdfw2, mdfb2,
      scw, scb, scg, scbb, mdw1, mdb1, mdg, mdbb, mdw2, mdb2, pcd)


def _row(x):
    return jnp.asarray(x, jnp.float32).reshape(1, -1)


def _pad_lane(x, n):
    return jnp.pad(jnp.asarray(x, jnp.float32), ((0, 0), (0, n - x.shape[1])))


def kernel(pcd_prev, K_prev, e_p, e_x, e_o, params):
    P = params
    del e_p, e_x, e_o

    k, xq, xk, xv = _front(
        K_prev, P['l1_w'], _row(P['l1_b']), _row(P['bn1_g']), _row(P['bn1_b']),
        P['st_lq_w'], _row(P['st_lq_b']), P['st_lk_w'], _row(P['st_lk_b']),
        P['st_lv_w'], _row(P['st_lv_b']))

    p0 = jnp.pad(pcd_prev[:_SEG], ((0, _SEGP - _SEG), (0, 0)))
    p1 = jnp.pad(pcd_prev[_SEG:], ((0, _SEGP - _SEG), (0, 0)))
    idx0 = _knn_segment(p0)
    idx1 = _knn_segment(p1) + _SEG
    idx2 = jnp.concatenate([idx0, idx1], axis=0).reshape(_ROWS // 128, 128)

    kv = jnp.concatenate([xk, xv], axis=1)
    pp128 = jnp.pad(pcd_prev, ((0, 0), (0, 125)))
    kvg, pgf = _gather2(kv, pp128, idx2)
    pg = pgf[:, :8]

    prep = jnp.repeat(pp128[:, :8], _NS, axis=0)

    lpw1 = jnp.zeros((8, 8), jnp.float32).at[:3, :3].set(P['st_lp_w1'])
    lpb1 = jnp.zeros((1, 8), jnp.float32).at[:, :3].set(P['st_lp_b1'])
    lpg = jnp.zeros((1, 8), jnp.float32).at[:, :3].set(P['st_lp_g'])
    lpbb = jnp.zeros((1, 8), jnp.float32).at[:, :3].set(P['st_lp_bb'])
    lpw2 = jnp.zeros((8, 128), jnp.float32).at[:3, :].set(P['st_lp_w2'])
    lpb2 = _row(P['st_lp_b2'])

    prs, prs2 = _prstats(pg, prep, lpw1, lpb1)

    wp, ws, ws2 = _wpre(kvg, xq, pg, prep, lpw1, lpb1, prs, prs2,
                        lpg, lpbb, lpw2, lpb2)

    h1, hs, hs2 = _h1(wp, ws, ws2, _row(P['st_lw_g0']), _row(P['st_lw_b0']),
                      P['st_lw_w1'], _row(P['st_lw_b1']))

    st, ss, ss2 = _agg(h1, hs, hs2, _row(P['st_lw_g1']), _row(P['st_lw_bb1']),
                       P['st_lw_w2'], _row(P['st_lw_b2']),
                       kvg, pg, prep, lpw1, lpb1, prs, prs2,
                       lpg, lpbb, lpw2, lpb2)

    pc_cat, kc_cat = _tail(
        st, k, ss, ss2, _row(P['bn2_g']), _row(P['bn2_b']),
        P['ps_w1'], _row(P['ps_b1']), _row(P['ps_g']), _row(P['ps_bb']),
        P['ps_w2'], _row(P['ps_b2']),
        P['psconv_w'][:, :, 0], P['psconv_w'][:, :, 1],
        P['mdf_w1'], _row(P['mdf_b1']), _row(P['mdf_g']), _row(P['mdf_bb']),
        P['mdf_w2'], _row(P['mdf_b2']),
        P['sc_w'], _row(P['sc_b']), _row(P['sc_g']), _row(P['sc_bb']),
        P['md_w1'], _row(P['md_b1']), _row(P['md_g']), _row(P['md_bb']),
        P['md_w2'], _row(P['md_b2']), pcd_prev)

    pcd_child = pc_cat.reshape(2, _N, 3).transpose(1, 0, 2).reshape(2 * _N, 3)
    k_curr = kc_cat.reshape(2, _N, 128).transpose(1, 0, 2).reshape(2 * _N, 128)
    return pcd_child, k_curr
